# Initial kernel scaffold; baseline (speedup 1.0000x reference)
import jax
import jax.numpy as jnp
from jax.experimental import pallas as pl


def kernel(city, table):
    raise NotImplementedError("write your pallas kernel here")


# speedup vs baseline: 1.2177x; 1.2177x over previous
import functools

import jax
import jax.numpy as jnp
from jax import lax
from jax.experimental import pallas as pl
from jax.experimental.pallas import tpu as pltpu, tpu_sc as plsc

_info = plsc.get_sparse_core_info()
_NC, _NS = _info.num_cores, _info.num_subcores
_NW = _NC * _NS

_CHUNK = 128


def _embed_lookup(city_eo, pair_table):
    n_pairs = city_eo.shape[2]
    n_chunks = n_pairs // _CHUNK
    mesh = plsc.VectorSubcoreMesh(core_axis_name="c", subcore_axis_name="s")

    @functools.partial(
        pl.kernel,
        mesh=mesh,
        out_type=jax.ShapeDtypeStruct((_NW, n_chunks, _CHUNK, 128), jnp.float32),
        scratch_types=[
            pltpu.VMEM((2, n_pairs), jnp.int32),
            pltpu.VMEM((n_chunks, _CHUNK), jnp.int32),
            pltpu.VMEM((n_chunks, _CHUNK, 128), jnp.float32),
            pltpu.SemaphoreType.DMA,
        ],
    )
    def k(ptab_hbm, idx_hbm, out_hbm, idx_v, pair_v, rows_v, sem):
        wid = lax.axis_index("s") * _NC + lax.axis_index("c")
        pltpu.sync_copy(idx_hbm.at[wid], idx_v)
        for g in range(n_pairs // 16):
            even = idx_v[0, pl.ds(g * 16, 16)]
            odd = idx_v[1, pl.ds(g * 16, 16)]
            r, c = divmod(g, _CHUNK // 16)
            pair_v[r, pl.ds(c * 16, 16)] = even * 5 + odd
        copies = [
            pltpu.async_copy(ptab_hbm.at[pair_v.at[j]], rows_v.at[j], sem)
            for j in range(n_chunks)
        ]
        for cp in copies:
            cp.wait()
        pltpu.sync_copy(rows_v, out_hbm.at[wid])

    return k(pair_table, city_eo)


def kernel(city, table):
    b = city.shape[0]
    d = table.shape[1]
    v = table.shape[0]
    left = jnp.repeat(table, v, axis=0)
    right = jnp.tile(table, (v, 1))
    pair_table = jnp.concatenate([left, right], axis=1)
    pair_table = jnp.pad(pair_table, ((0, 32 - v * v), (0, 0)))
    c3 = city.astype(jnp.int32).reshape(_NW, b // (2 * _NW), 2)
    city_eo = jnp.stack([c3[:, :, 0], c3[:, :, 1]], axis=1)
    out = _embed_lookup(city_eo, pair_table)
    return out.reshape(b, d)
